# baseline (speedup 1.0000x reference)
import jax
import jax.numpy as jnp
from jax import lax
from jax.experimental import pallas as pl
from jax.experimental.pallas import tpu as pltpu
from jax.experimental.pallas import tpu_sc as plsc

N = 100000
H = 128
NUM_VN = 4
NUM_HEADS = 4
HEAD_DIM = H // NUM_HEADS
B = 256
TBL = B * NUM_VN

NP = 102400
BA = 2048
NBLKA = NP // BA
LASTA = N // BA
BN = 2000
NBLK = N // BN

NC = 2
NS = 16
NW = NC * NS
ROWS_W = NP // NW
CH = 128
NCH = ROWS_W // CH


def _stage_a(x_ref, p_ref, b_ref, w1t_ref, b1_ref, w2_ref, b2_ref,
             mw_ref, mb_ref, r2v_ref, flat_ref):
    pid = pl.program_id(0)
    x = x_ref[...]
    pri = p_ref[...]
    x129 = jnp.concatenate([x, pri], axis=1)
    h1 = jnp.dot(x129, w1t_ref[...], preferred_element_type=jnp.float32)
    h1 = jnp.maximum(h1 + b1_ref[...], 0.0)
    logits = jnp.dot(h1, w2_ref[...], preferred_element_type=jnp.float32)
    logits = logits + b2_ref[...]
    m = jnp.min(logits, axis=1, keepdims=True)
    iota = lax.broadcasted_iota(jnp.int32, (BA, NUM_VN), 1)
    e = jnp.max(jnp.where(logits <= m, iota, -1), axis=1, keepdims=True)
    flat = jnp.clip(b_ref[...] * NUM_VN + e, 0, TBL - 1)
    flat_ref[...] = flat
    r2v = jnp.dot(x, mw_ref[...], preferred_element_type=jnp.float32,
                  precision=lax.Precision.HIGHEST)
    r2v = r2v + mb_ref[...]
    row = pid * BA + lax.broadcasted_iota(jnp.int32, (BA, 1), 0)
    r2v_ref[...] = jnp.where(row < N, r2v, 0.0)


def _run_stage_a(real_nodes, priors, batch2d, w1t, b1, w2t, b2, mwt, mb):
    def node_map(i):
        return (jnp.minimum(i, LASTA), 0)

    return pl.pallas_call(
        _stage_a,
        grid=(NBLKA,),
        in_specs=[
            pl.BlockSpec((BA, H), node_map),
            pl.BlockSpec((BA, 1), node_map),
            pl.BlockSpec((BA, 1), node_map),
            pl.BlockSpec((H + 1, H), lambda i: (0, 0)),
            pl.BlockSpec((1, H), lambda i: (0, 0)),
            pl.BlockSpec((H, NUM_VN), lambda i: (0, 0)),
            pl.BlockSpec((1, NUM_VN), lambda i: (0, 0)),
            pl.BlockSpec((H, H), lambda i: (0, 0)),
            pl.BlockSpec((1, H), lambda i: (0, 0)),
        ],
        out_specs=[
            pl.BlockSpec((BA, H), lambda i: (i, 0)),
            pl.BlockSpec((BA, 1), lambda i: (i, 0)),
        ],
        out_shape=[
            jax.ShapeDtypeStruct((NP, H), jnp.float32),
            jax.ShapeDtypeStruct((NP, 1), jnp.int32),
        ],
    )(real_nodes, priors, batch2d, w1t, b1, w2t, b2, mwt, mb)


def _sc_scatter(r2v_hbm, idx_hbm, zeros_hbm, out_hbm, data_v, idx_v, table_sh):
    c = lax.axis_index("c")
    s = lax.axis_index("s")
    wid = s * NC + c
    rows_per_tile = TBL // NS
    pltpu.sync_copy(zeros_hbm.at[pl.ds(s * rows_per_tile, rows_per_tile)],
                    table_sh.at[pl.ds(s * rows_per_tile, rows_per_tile)])
    pltpu.sync_copy(idx_hbm.at[wid], idx_v)
    plsc.subcore_barrier()

    def body(t, carry):
        base = wid * ROWS_W + t * CH
        pltpu.sync_copy(r2v_hbm.at[pl.ds(base, CH)], data_v)
        pltpu.sync_copy(data_v, table_sh.at[idx_v.at[t]], add=True)
        return carry

    lax.fori_loop(0, NCH, body, 0)
    plsc.subcore_barrier()
    pltpu.sync_copy(table_sh.at[pl.ds(s * rows_per_tile, rows_per_tile)],
                    out_hbm.at[c, pl.ds(s * rows_per_tile, rows_per_tile)])


def _run_stage_b(r2v, idx3d, zeros_tbl):
    mesh = plsc.VectorSubcoreMesh(core_axis_name="c", subcore_axis_name="s",
                                  num_cores=NC, num_subcores=NS)
    f = pl.kernel(
        _sc_scatter,
        mesh=mesh,
        out_type=jax.ShapeDtypeStruct((NC, TBL, H), jnp.float32),
        scratch_types=[
            pltpu.VMEM((CH, H), jnp.float32),
            pltpu.VMEM((NCH, CH), jnp.int32),
            pltpu.MemorySpace.VMEM_SHARED((TBL, H), jnp.float32),
        ],
    )
    return f(r2v, idx3d, zeros_tbl)


def _stage_c(part_ref, vn0_ref, wi_ref, bi_ref, wo_ref, bo_ref, out_ref):
    tg = part_ref[0] + part_ref[1]
    s4 = tg[:, 0, :] + tg[:, 1, :] + tg[:, 2, :] + tg[:, 3, :]
    vn = [vn0_ref[0, i, :][None, :] + (s4 - tg[:, i, :]) for i in range(NUM_VN)]
    qkv = [jnp.dot(vn[i], wi_ref[...], preferred_element_type=jnp.float32,
                   precision=lax.Precision.HIGHEST)
           + bi_ref[...] for i in range(NUM_VN)]
    q = [t[:, :H] for t in qkv]
    k = [t[:, H:2 * H] for t in qkv]
    v = [t[:, 2 * H:] for t in qkv]
    di = lax.broadcasted_iota(jnp.int32, (H, NUM_HEADS), 0)
    hi = lax.broadcasted_iota(jnp.int32, (H, NUM_HEADS), 1)
    hm = (di // HEAD_DIM == hi).astype(jnp.float32)
    scale = 1.0 / (HEAD_DIM ** 0.5)
    out_vn = []
    for i in range(NUM_VN):
        s_ij = [jnp.dot(q[i] * k[j], hm, preferred_element_type=jnp.float32,
                        precision=lax.Precision.HIGHEST)
                * scale for j in range(NUM_VN)]
        mx = jnp.maximum(jnp.maximum(s_ij[0], s_ij[1]),
                         jnp.maximum(s_ij[2], s_ij[3]))
        ex = [jnp.exp(t - mx) for t in s_ij]
        den = ex[0] + ex[1] + ex[2] + ex[3]
        o = jnp.zeros((B, H), jnp.float32)
        for j in range(NUM_VN):
            a_bc = jnp.dot(ex[j] / den, hm.T,
                           preferred_element_type=jnp.float32,
                           precision=lax.Precision.HIGHEST)
            o = o + a_bc * v[j]
        attn = jnp.dot(o, wo_ref[...], preferred_element_type=jnp.float32,
                       precision=lax.Precision.HIGHEST)
        out_vn.append(vn[i] + attn + bo_ref[...])
    ss = out_vn[0] + out_vn[1] + out_vn[2] + out_vn[3]
    for i in range(NUM_VN):
        out_ref[:, i, :] = ss - out_vn[i]


def _run_stage_c(parts, vn_init, wit, bi, wot, bo):
    return pl.pallas_call(
        _stage_c,
        out_shape=jax.ShapeDtypeStruct((B, NUM_VN, H), jnp.float32),
    )(parts, vn_init, wit, bi, wot, bo)


def _sc_gather(tbl_hbm, idx_hbm, out_hbm, idx_v, rows_v, table_sh, sem):
    c = lax.axis_index("c")
    s = lax.axis_index("s")
    wid = s * NC + c
    rows_per_tile = TBL // NS
    pltpu.sync_copy(tbl_hbm.at[pl.ds(s * rows_per_tile, rows_per_tile)],
                    table_sh.at[pl.ds(s * rows_per_tile, rows_per_tile)])
    pltpu.sync_copy(idx_hbm.at[wid], idx_v)
    plsc.subcore_barrier()

    def body(t, carry):
        base = wid * ROWS_W + t * CH
        pltpu.async_copy(table_sh.at[idx_v.at[t]], rows_v, sem).wait()
        pltpu.sync_copy(rows_v, out_hbm.at[pl.ds(base, CH)])
        return carry

    lax.fori_loop(0, NCH, body, 0)


def _run_stage_d(tbl, idx3d):
    mesh = plsc.VectorSubcoreMesh(core_axis_name="c", subcore_axis_name="s",
                                  num_cores=NC, num_subcores=NS)
    f = pl.kernel(
        _sc_gather,
        mesh=mesh,
        out_type=jax.ShapeDtypeStruct((NP, H), jnp.float32),
        scratch_types=[
            pltpu.VMEM((NCH, CH), jnp.int32),
            pltpu.VMEM((CH, H), jnp.float32),
            pltpu.MemorySpace.VMEM_SHARED((TBL, H), jnp.float32),
            pltpu.SemaphoreType.DMA,
        ],
    )
    return f(tbl, idx3d)


def _stage_e(x_ref, rm_ref, wih_ref, bih_ref, whh_ref, bhh_ref, out_ref):
    x = x_ref[...]
    rm = rm_ref[...]
    gi = jnp.dot(rm, wih_ref[...], preferred_element_type=jnp.float32,
                 precision=lax.Precision.HIGHEST)
    gi = gi + bih_ref[...]
    gh = jnp.dot(x.astype(jnp.bfloat16), whh_ref[...],
                 preferred_element_type=jnp.float32)
    gh = gh + bhh_ref[...]
    r = jax.nn.sigmoid(gi[:, :H] + gh[:, :H])
    z = jax.nn.sigmoid(gi[:, H:2 * H] + gh[:, H:2 * H])
    n = jnp.tanh(gi[:, 2 * H:] + r * gh[:, 2 * H:])
    out_ref[...] = (1.0 - z) * n + z * x


def _run_stage_e(real_nodes, real_msg, wiht, bih, whht, bhh):
    return pl.pallas_call(
        _stage_e,
        grid=(NBLK,),
        in_specs=[
            pl.BlockSpec((BN, H), lambda i: (i, 0)),
            pl.BlockSpec((BN, H), lambda i: (i, 0)),
            pl.BlockSpec((H, 3 * H), lambda i: (0, 0)),
            pl.BlockSpec((1, 3 * H), lambda i: (0, 0)),
            pl.BlockSpec((H, 3 * H), lambda i: (0, 0)),
            pl.BlockSpec((1, 3 * H), lambda i: (0, 0)),
        ],
        out_specs=pl.BlockSpec((BN, H), lambda i: (i, 0)),
        out_shape=jax.ShapeDtypeStruct((N, H), jnp.float32),
    )(real_nodes, real_msg, wiht, bih, whht, bhh)


def kernel(real_nodes, batch, priors, vn_init, router_w1, router_b1,
           router_w2, router_b2, msg_w, msg_b, attn_in_w, attn_in_b,
           attn_out_w, attn_out_b, gru_wih, gru_whh, gru_bih, gru_bhh):
    w1t = router_w1.T
    b1 = router_b1.reshape(1, H)
    w2t = router_w2.T
    b2 = router_b2.reshape(1, NUM_VN)
    mwt = msg_w.T
    mb = msg_b.reshape(1, H)
    batch2d = batch.reshape(N, 1)

    r2v, flat = _run_stage_a(real_nodes, priors, batch2d,
                             w1t, b1, w2t, b2, mwt, mb)
    idx3d = flat.reshape(NW, NCH, CH)
    zeros_tbl = jnp.zeros((TBL, H), jnp.float32)
    parts = _run_stage_b(r2v, idx3d, zeros_tbl)

    r_tbl = _run_stage_c(parts.reshape(NC, B, NUM_VN, H), vn_init,
                         attn_in_w.T, attn_in_b.reshape(1, 3 * H),
                         attn_out_w.T, attn_out_b.reshape(1, H))
    real_msg = _run_stage_d(r_tbl.reshape(TBL, H), idx3d)

    return _run_stage_e(real_nodes, real_msg,
                        gru_wih.T, gru_bih.reshape(1, 3 * H),
                        gru_whh.T.astype(jnp.bfloat16),
                        gru_bhh.reshape(1, 3 * H))


# speedup vs baseline: 1.3731x; 1.3731x over previous
import jax
import jax.numpy as jnp
from jax import lax
from jax.experimental import pallas as pl
from jax.experimental.pallas import tpu as pltpu
from jax.experimental.pallas import tpu_sc as plsc

N = 100000
H = 128
NUM_VN = 4
NUM_HEADS = 4
HEAD_DIM = H // NUM_HEADS
B = 256
TBL = B * NUM_VN

NP = 102400
BA = 5120
NBLKA = NP // BA
LASTA = N // BA
BN = 5000
NBLK = N // BN

NC = 2
NS = 16
NW = NC * NS
ROWS_W = NP // NW
CH = 128
NCH = ROWS_W // CH


def _stage_a(x_ref, p_ref, b_ref, w1t_ref, b1_ref, w2_ref, b2_ref,
             mw_ref, mb_ref, r2v_ref, flat_ref):
    pid = pl.program_id(0)
    x = x_ref[...]
    pri = p_ref[...]
    x129 = jnp.concatenate([x, pri], axis=1)
    h1 = jnp.dot(x129, w1t_ref[...], preferred_element_type=jnp.float32)
    h1 = jnp.maximum(h1 + b1_ref[...], 0.0)
    logits = jnp.dot(h1, w2_ref[...], preferred_element_type=jnp.float32)
    logits = logits + b2_ref[...]
    m = jnp.min(logits, axis=1, keepdims=True)
    iota = lax.broadcasted_iota(jnp.int32, (BA, NUM_VN), 1)
    e = jnp.max(jnp.where(logits <= m, iota, -1), axis=1, keepdims=True)
    flat = jnp.clip(b_ref[...] * NUM_VN + e, 0, TBL - 1)
    flat_ref[...] = flat
    r2v = jnp.dot(x, mw_ref[...], preferred_element_type=jnp.float32)
    r2v = r2v + mb_ref[...]
    row = pid * BA + lax.broadcasted_iota(jnp.int32, (BA, 1), 0)
    r2v_ref[...] = jnp.where(row < N, r2v, 0.0)


def _run_stage_a(real_nodes, priors, batch2d, w1t, b1, w2t, b2, mwt, mb):
    def node_map(i):
        return (jnp.minimum(i, LASTA), 0)

    return pl.pallas_call(
        _stage_a,
        grid=(NBLKA,),
        in_specs=[
            pl.BlockSpec((BA, H), node_map),
            pl.BlockSpec((BA, 1), node_map),
            pl.BlockSpec((BA, 1), node_map),
            pl.BlockSpec((H + 1, H), lambda i: (0, 0)),
            pl.BlockSpec((1, H), lambda i: (0, 0)),
            pl.BlockSpec((H, NUM_VN), lambda i: (0, 0)),
            pl.BlockSpec((1, NUM_VN), lambda i: (0, 0)),
            pl.BlockSpec((H, H), lambda i: (0, 0)),
            pl.BlockSpec((1, H), lambda i: (0, 0)),
        ],
        out_specs=[
            pl.BlockSpec((BA, H), lambda i: (i, 0)),
            pl.BlockSpec((BA, 1), lambda i: (i, 0)),
        ],
        out_shape=[
            jax.ShapeDtypeStruct((NP, H), jnp.float32),
            jax.ShapeDtypeStruct((NP, 1), jnp.int32),
        ],
    )(real_nodes, priors, batch2d, w1t, b1, w2t, b2, mwt, mb)


def _sc_scatter(r2v_hbm, idx_hbm, zeros_hbm, out_hbm, data_v, idx_v, table_sh):
    c = lax.axis_index("c")
    s = lax.axis_index("s")
    wid = s * NC + c
    rows_per_tile = TBL // NS
    pltpu.sync_copy(zeros_hbm.at[pl.ds(s * rows_per_tile, rows_per_tile)],
                    table_sh.at[pl.ds(s * rows_per_tile, rows_per_tile)])
    pltpu.sync_copy(idx_hbm.at[wid], idx_v)
    plsc.subcore_barrier()

    def body(t, carry):
        base = wid * ROWS_W + t * CH
        pltpu.sync_copy(r2v_hbm.at[pl.ds(base, CH)], data_v)
        pltpu.sync_copy(data_v, table_sh.at[idx_v.at[t]], add=True)
        return carry

    lax.fori_loop(0, NCH, body, 0)
    plsc.subcore_barrier()
    pltpu.sync_copy(table_sh.at[pl.ds(s * rows_per_tile, rows_per_tile)],
                    out_hbm.at[c, pl.ds(s * rows_per_tile, rows_per_tile)])


def _run_stage_b(r2v, idx3d, zeros_tbl):
    mesh = plsc.VectorSubcoreMesh(core_axis_name="c", subcore_axis_name="s",
                                  num_cores=NC, num_subcores=NS)
    f = pl.kernel(
        _sc_scatter,
        mesh=mesh,
        out_type=jax.ShapeDtypeStruct((NC, TBL, H), jnp.float32),
        scratch_types=[
            pltpu.VMEM((CH, H), jnp.float32),
            pltpu.VMEM((NCH, CH), jnp.int32),
            pltpu.MemorySpace.VMEM_SHARED((TBL, H), jnp.float32),
        ],
    )
    return f(r2v, idx3d, zeros_tbl)


def _stage_c(part_ref, vn0_ref, wi_ref, bi_ref, wo_ref, bo_ref, out_ref):
    tg = part_ref[0] + part_ref[1]
    s4 = tg[:, 0, :] + tg[:, 1, :] + tg[:, 2, :] + tg[:, 3, :]
    vn = [vn0_ref[0, i, :][None, :] + (s4 - tg[:, i, :]) for i in range(NUM_VN)]
    qkv = [jnp.dot(vn[i], wi_ref[...], preferred_element_type=jnp.float32)
           + bi_ref[...] for i in range(NUM_VN)]
    q = [t[:, :H] for t in qkv]
    k = [t[:, H:2 * H] for t in qkv]
    v = [t[:, 2 * H:] for t in qkv]
    di = lax.broadcasted_iota(jnp.int32, (H, NUM_HEADS), 0)
    hi = lax.broadcasted_iota(jnp.int32, (H, NUM_HEADS), 1)
    hm = (di // HEAD_DIM == hi).astype(jnp.float32)
    scale = 1.0 / (HEAD_DIM ** 0.5)
    out_vn = []
    for i in range(NUM_VN):
        s_ij = [jnp.dot(q[i] * k[j], hm, preferred_element_type=jnp.float32)
                * scale for j in range(NUM_VN)]
        mx = jnp.maximum(jnp.maximum(s_ij[0], s_ij[1]),
                         jnp.maximum(s_ij[2], s_ij[3]))
        ex = [jnp.exp(t - mx) for t in s_ij]
        den = ex[0] + ex[1] + ex[2] + ex[3]
        o = jnp.zeros((B, H), jnp.float32)
        for j in range(NUM_VN):
            a_bc = jnp.dot(ex[j] / den, hm.T,
                           preferred_element_type=jnp.float32)
            o = o + a_bc * v[j]
        attn = jnp.dot(o, wo_ref[...], preferred_element_type=jnp.float32)
        out_vn.append(vn[i] + attn + bo_ref[...])
    ss = out_vn[0] + out_vn[1] + out_vn[2] + out_vn[3]
    for i in range(NUM_VN):
        out_ref[:, i, :] = ss - out_vn[i]


def _run_stage_c(parts, vn_init, wit, bi, wot, bo):
    return pl.pallas_call(
        _stage_c,
        out_shape=jax.ShapeDtypeStruct((B, NUM_VN, H), jnp.float32),
    )(parts, vn_init, wit, bi, wot, bo)


def _sc_gather(tbl_hbm, idx_hbm, out_hbm, idx_v, rows_v, table_sh, sem):
    c = lax.axis_index("c")
    s = lax.axis_index("s")
    wid = s * NC + c
    rows_per_tile = TBL // NS
    pltpu.sync_copy(tbl_hbm.at[pl.ds(s * rows_per_tile, rows_per_tile)],
                    table_sh.at[pl.ds(s * rows_per_tile, rows_per_tile)])
    pltpu.sync_copy(idx_hbm.at[wid], idx_v)
    plsc.subcore_barrier()

    def body(t, carry):
        base = wid * ROWS_W + t * CH
        pltpu.async_copy(table_sh.at[idx_v.at[t]], rows_v, sem).wait()
        pltpu.sync_copy(rows_v, out_hbm.at[pl.ds(base, CH)])
        return carry

    lax.fori_loop(0, NCH, body, 0)


def _run_stage_d(tbl, idx3d):
    mesh = plsc.VectorSubcoreMesh(core_axis_name="c", subcore_axis_name="s",
                                  num_cores=NC, num_subcores=NS)
    f = pl.kernel(
        _sc_gather,
        mesh=mesh,
        out_type=jax.ShapeDtypeStruct((NP, H), jnp.float32),
        scratch_types=[
            pltpu.VMEM((NCH, CH), jnp.int32),
            pltpu.VMEM((CH, H), jnp.float32),
            pltpu.MemorySpace.VMEM_SHARED((TBL, H), jnp.float32),
            pltpu.SemaphoreType.DMA,
        ],
    )
    return f(tbl, idx3d)


def _stage_e(x_ref, rm_ref, wih_ref, bih_ref, whh_ref, bhh_ref, out_ref):
    x = x_ref[...]
    rm = rm_ref[...]
    gi = jnp.dot(rm, wih_ref[...], preferred_element_type=jnp.float32)
    gi = gi + bih_ref[...]
    gh = jnp.dot(x, whh_ref[...], preferred_element_type=jnp.float32)
    gh = gh + bhh_ref[...]
    r = jax.nn.sigmoid(gi[:, :H] + gh[:, :H])
    z = jax.nn.sigmoid(gi[:, H:2 * H] + gh[:, H:2 * H])
    n = jnp.tanh(gi[:, 2 * H:] + r * gh[:, 2 * H:])
    out_ref[...] = (1.0 - z) * n + z * x


def _run_stage_e(real_nodes, real_msg, wiht, bih, whht, bhh):
    return pl.pallas_call(
        _stage_e,
        grid=(NBLK,),
        in_specs=[
            pl.BlockSpec((BN, H), lambda i: (i, 0)),
            pl.BlockSpec((BN, H), lambda i: (i, 0)),
            pl.BlockSpec((H, 3 * H), lambda i: (0, 0)),
            pl.BlockSpec((1, 3 * H), lambda i: (0, 0)),
            pl.BlockSpec((H, 3 * H), lambda i: (0, 0)),
            pl.BlockSpec((1, 3 * H), lambda i: (0, 0)),
        ],
        out_specs=pl.BlockSpec((BN, H), lambda i: (i, 0)),
        out_shape=jax.ShapeDtypeStruct((N, H), jnp.float32),
    )(real_nodes, real_msg, wiht, bih, whht, bhh)


def kernel(real_nodes, batch, priors, vn_init, router_w1, router_b1,
           router_w2, router_b2, msg_w, msg_b, attn_in_w, attn_in_b,
           attn_out_w, attn_out_b, gru_wih, gru_whh, gru_bih, gru_bhh):
    w1t = router_w1.T
    b1 = router_b1.reshape(1, H)
    w2t = router_w2.T
    b2 = router_b2.reshape(1, NUM_VN)
    mwt = msg_w.T
    mb = msg_b.reshape(1, H)
    batch2d = batch.reshape(N, 1)

    r2v, flat = _run_stage_a(real_nodes, priors, batch2d,
                             w1t, b1, w2t, b2, mwt, mb)
    idx3d = flat.reshape(NW, NCH, CH)
    zeros_tbl = jnp.zeros((TBL, H), jnp.float32)
    parts = _run_stage_b(r2v, idx3d, zeros_tbl)

    r_tbl = _run_stage_c(parts.reshape(NC, B, NUM_VN, H), vn_init,
                         attn_in_w.T, attn_in_b.reshape(1, 3 * H),
                         attn_out_w.T, attn_out_b.reshape(1, H))
    real_msg = _run_stage_d(r_tbl.reshape(TBL, H), idx3d)

    return _run_stage_e(real_nodes, real_msg,
                        gru_wih.T, gru_bih.reshape(1, 3 * H),
                        gru_whh.T, gru_bhh.reshape(1, 3 * H))


# speedup vs baseline: 1.4031x; 1.0218x over previous
import jax
import jax.numpy as jnp
from jax import lax
from jax.experimental import pallas as pl
from jax.experimental.pallas import tpu as pltpu
from jax.experimental.pallas import tpu_sc as plsc

N = 100000
H = 128
NUM_VN = 4
NUM_HEADS = 4
HEAD_DIM = H // NUM_HEADS
B = 256
TBL = B * NUM_VN

NP = 102400
BA = 10240
NBLKA = NP // BA
LASTA = N // BA
BN = 10000
NBLK = N // BN

NC = 2
NS = 16
NW = NC * NS
ROWS_W = NP // NW
CH = 128
NCH = ROWS_W // CH


def _stage_a(x_ref, p_ref, b_ref, w1t_ref, b1_ref, w2_ref, b2_ref,
             mw_ref, mb_ref, r2v_ref, flat_ref):
    pid = pl.program_id(0)
    x = x_ref[...]
    pri = p_ref[...]
    x129 = jnp.concatenate([x, pri], axis=1)
    h1 = jnp.dot(x129, w1t_ref[...], preferred_element_type=jnp.float32)
    h1 = jnp.maximum(h1 + b1_ref[...], 0.0)
    logits = jnp.dot(h1, w2_ref[...], preferred_element_type=jnp.float32)
    logits = logits + b2_ref[...]
    m = jnp.min(logits, axis=1, keepdims=True)
    iota = lax.broadcasted_iota(jnp.int32, (BA, NUM_VN), 1)
    e = jnp.max(jnp.where(logits <= m, iota, -1), axis=1, keepdims=True)
    flat = jnp.clip(b_ref[...] * NUM_VN + e, 0, TBL - 1)
    flat_ref[...] = flat
    r2v = jnp.dot(x, mw_ref[...], preferred_element_type=jnp.float32)
    r2v = r2v + mb_ref[...]
    row = pid * BA + lax.broadcasted_iota(jnp.int32, (BA, 1), 0)
    r2v_ref[...] = jnp.where(row < N, r2v, 0.0)


def _run_stage_a(real_nodes, priors, batch2d, w1t, b1, w2t, b2, mwt, mb):
    def node_map(i):
        return (jnp.minimum(i, LASTA), 0)

    return pl.pallas_call(
        _stage_a,
        grid=(NBLKA,),
        in_specs=[
            pl.BlockSpec((BA, H), node_map),
            pl.BlockSpec((BA, 1), node_map),
            pl.BlockSpec((BA, 1), node_map),
            pl.BlockSpec((H + 1, H), lambda i: (0, 0)),
            pl.BlockSpec((1, H), lambda i: (0, 0)),
            pl.BlockSpec((H, NUM_VN), lambda i: (0, 0)),
            pl.BlockSpec((1, NUM_VN), lambda i: (0, 0)),
            pl.BlockSpec((H, H), lambda i: (0, 0)),
            pl.BlockSpec((1, H), lambda i: (0, 0)),
        ],
        out_specs=[
            pl.BlockSpec((BA, H), lambda i: (i, 0)),
            pl.BlockSpec((BA, 1), lambda i: (i, 0)),
        ],
        out_shape=[
            jax.ShapeDtypeStruct((NP, H), jnp.float32),
            jax.ShapeDtypeStruct((NP, 1), jnp.int32),
        ],
    )(real_nodes, priors, batch2d, w1t, b1, w2t, b2, mwt, mb)


def _sc_scatter(r2v_hbm, idx_hbm, zeros_hbm, out_hbm, data_v, idx_v, table_sh):
    c = lax.axis_index("c")
    s = lax.axis_index("s")
    wid = s * NC + c
    rows_per_tile = TBL // NS
    pltpu.sync_copy(zeros_hbm.at[pl.ds(s * rows_per_tile, rows_per_tile)],
                    table_sh.at[pl.ds(s * rows_per_tile, rows_per_tile)])
    pltpu.sync_copy(idx_hbm.at[wid], idx_v)
    plsc.subcore_barrier()

    def body(t, carry):
        base = wid * ROWS_W + t * CH
        pltpu.sync_copy(r2v_hbm.at[pl.ds(base, CH)], data_v)
        pltpu.sync_copy(data_v, table_sh.at[idx_v.at[t]], add=True)
        return carry

    lax.fori_loop(0, NCH, body, 0)
    plsc.subcore_barrier()
    pltpu.sync_copy(table_sh.at[pl.ds(s * rows_per_tile, rows_per_tile)],
                    out_hbm.at[c, pl.ds(s * rows_per_tile, rows_per_tile)])


def _run_stage_b(r2v, idx3d, zeros_tbl):
    mesh = plsc.VectorSubcoreMesh(core_axis_name="c", subcore_axis_name="s",
                                  num_cores=NC, num_subcores=NS)
    f = pl.kernel(
        _sc_scatter,
        mesh=mesh,
        out_type=jax.ShapeDtypeStruct((NC, TBL, H), jnp.float32),
        scratch_types=[
            pltpu.VMEM((CH, H), jnp.float32),
            pltpu.VMEM((NCH, CH), jnp.int32),
            pltpu.MemorySpace.VMEM_SHARED((TBL, H), jnp.float32),
        ],
    )
    return f(r2v, idx3d, zeros_tbl)


def _stage_c(part_ref, vn0_ref, wi_ref, bi_ref, wo_ref, bo_ref, out_ref):
    tg = part_ref[0] + part_ref[1]
    s4 = tg[:, 0, :] + tg[:, 1, :] + tg[:, 2, :] + tg[:, 3, :]
    vn = [vn0_ref[0, i, :][None, :] + (s4 - tg[:, i, :]) for i in range(NUM_VN)]
    qkv = [jnp.dot(vn[i], wi_ref[...], preferred_element_type=jnp.float32)
           + bi_ref[...] for i in range(NUM_VN)]
    q = [t[:, :H] for t in qkv]
    k = [t[:, H:2 * H] for t in qkv]
    v = [t[:, 2 * H:] for t in qkv]
    di = lax.broadcasted_iota(jnp.int32, (H, NUM_HEADS), 0)
    hi = lax.broadcasted_iota(jnp.int32, (H, NUM_HEADS), 1)
    hm = (di // HEAD_DIM == hi).astype(jnp.float32)
    scale = 1.0 / (HEAD_DIM ** 0.5)
    out_vn = []
    for i in range(NUM_VN):
        s_ij = [jnp.dot(q[i] * k[j], hm, preferred_element_type=jnp.float32)
                * scale for j in range(NUM_VN)]
        mx = jnp.maximum(jnp.maximum(s_ij[0], s_ij[1]),
                         jnp.maximum(s_ij[2], s_ij[3]))
        ex = [jnp.exp(t - mx) for t in s_ij]
        den = ex[0] + ex[1] + ex[2] + ex[3]
        o = jnp.zeros((B, H), jnp.float32)
        for j in range(NUM_VN):
            a_bc = jnp.dot(ex[j] / den, hm.T,
                           preferred_element_type=jnp.float32)
            o = o + a_bc * v[j]
        attn = jnp.dot(o, wo_ref[...], preferred_element_type=jnp.float32)
        out_vn.append(vn[i] + attn + bo_ref[...])
    ss = out_vn[0] + out_vn[1] + out_vn[2] + out_vn[3]
    for i in range(NUM_VN):
        out_ref[:, i, :] = ss - out_vn[i]


def _run_stage_c(parts, vn_init, wit, bi, wot, bo):
    return pl.pallas_call(
        _stage_c,
        out_shape=jax.ShapeDtypeStruct((B, NUM_VN, H), jnp.float32),
    )(parts, vn_init, wit, bi, wot, bo)


def _sc_gather(tbl_hbm, idx_hbm, out_hbm, idx_v, rows_v, table_sh, sem):
    c = lax.axis_index("c")
    s = lax.axis_index("s")
    wid = s * NC + c
    rows_per_tile = TBL // NS
    pltpu.sync_copy(tbl_hbm.at[pl.ds(s * rows_per_tile, rows_per_tile)],
                    table_sh.at[pl.ds(s * rows_per_tile, rows_per_tile)])
    pltpu.sync_copy(idx_hbm.at[wid], idx_v)
    plsc.subcore_barrier()

    def body(t, carry):
        base = wid * ROWS_W + t * CH
        pltpu.async_copy(table_sh.at[idx_v.at[t]], rows_v, sem).wait()
        pltpu.sync_copy(rows_v, out_hbm.at[pl.ds(base, CH)])
        return carry

    lax.fori_loop(0, NCH, body, 0)


def _run_stage_d(tbl, idx3d):
    mesh = plsc.VectorSubcoreMesh(core_axis_name="c", subcore_axis_name="s",
                                  num_cores=NC, num_subcores=NS)
    f = pl.kernel(
        _sc_gather,
        mesh=mesh,
        out_type=jax.ShapeDtypeStruct((NP, H), jnp.float32),
        scratch_types=[
            pltpu.VMEM((NCH, CH), jnp.int32),
            pltpu.VMEM((CH, H), jnp.float32),
            pltpu.MemorySpace.VMEM_SHARED((TBL, H), jnp.float32),
            pltpu.SemaphoreType.DMA,
        ],
    )
    return f(tbl, idx3d)


def _stage_e(x_ref, rm_ref, wih_ref, bih_ref, whh_ref, bhh_ref, out_ref):
    x = x_ref[...]
    rm = rm_ref[...]
    gi = jnp.dot(rm, wih_ref[...], preferred_element_type=jnp.float32)
    gi = gi + bih_ref[...]
    gh = jnp.dot(x, whh_ref[...], preferred_element_type=jnp.float32)
    gh = gh + bhh_ref[...]
    r = jax.nn.sigmoid(gi[:, :H] + gh[:, :H])
    z = jax.nn.sigmoid(gi[:, H:2 * H] + gh[:, H:2 * H])
    n = jnp.tanh(gi[:, 2 * H:] + r * gh[:, 2 * H:])
    out_ref[...] = (1.0 - z) * n + z * x


def _run_stage_e(real_nodes, real_msg, wiht, bih, whht, bhh):
    return pl.pallas_call(
        _stage_e,
        grid=(NBLK,),
        in_specs=[
            pl.BlockSpec((BN, H), lambda i: (i, 0)),
            pl.BlockSpec((BN, H), lambda i: (i, 0)),
            pl.BlockSpec((H, 3 * H), lambda i: (0, 0)),
            pl.BlockSpec((1, 3 * H), lambda i: (0, 0)),
            pl.BlockSpec((H, 3 * H), lambda i: (0, 0)),
            pl.BlockSpec((1, 3 * H), lambda i: (0, 0)),
        ],
        out_specs=pl.BlockSpec((BN, H), lambda i: (i, 0)),
        out_shape=jax.ShapeDtypeStruct((N, H), jnp.float32),
    )(real_nodes, real_msg, wiht, bih, whht, bhh)


def kernel(real_nodes, batch, priors, vn_init, router_w1, router_b1,
           router_w2, router_b2, msg_w, msg_b, attn_in_w, attn_in_b,
           attn_out_w, attn_out_b, gru_wih, gru_whh, gru_bih, gru_bhh):
    w1t = router_w1.T
    b1 = router_b1.reshape(1, H)
    w2t = router_w2.T
    b2 = router_b2.reshape(1, NUM_VN)
    mwt = msg_w.T
    mb = msg_b.reshape(1, H)
    batch2d = batch.reshape(N, 1)

    r2v, flat = _run_stage_a(real_nodes, priors, batch2d,
                             w1t, b1, w2t, b2, mwt, mb)
    idx3d = flat.reshape(NW, NCH, CH)
    zeros_tbl = jnp.zeros((TBL, H), jnp.float32)
    parts = _run_stage_b(r2v, idx3d, zeros_tbl)

    r_tbl = _run_stage_c(parts.reshape(NC, B, NUM_VN, H), vn_init,
                         attn_in_w.T, attn_in_b.reshape(1, 3 * H),
                         attn_out_w.T, attn_out_b.reshape(1, H))
    real_msg = _run_stage_d(r_tbl.reshape(TBL, H), idx3d)

    return _run_stage_e(real_nodes, real_msg,
                        gru_wih.T, gru_bih.reshape(1, 3 * H),
                        gru_whh.T, gru_bhh.reshape(1, 3 * H))
